# Initial kernel scaffold; baseline (speedup 1.0000x reference)
import jax
import jax.numpy as jnp
from jax.experimental import pallas as pl


def kernel(x, edge_index, attention_weights, W, b):
    raise NotImplementedError("write your pallas kernel here")


# speedup vs baseline: 26.8502x; 26.8502x over previous
import functools

import jax
import jax.numpy as jnp
from jax import lax
from jax.experimental import pallas as pl
from jax.experimental.pallas import tpu as pltpu
from jax.experimental.pallas import tpu_sc as plsc

N = 10000
E = 320000
D = 128

NC = 2
NS = 16
NW = NC * NS
EPW = E // NW
CH = 80
NCHUNK = EPW // CH

NPAD = 10240
RPT = NPAD // NS

_mesh = plsc.VectorSubcoreMesh(core_axis_name="c", subcore_axis_name="s")

_Z16 = functools.partial(jnp.zeros, (16,), jnp.float32)


@functools.partial(
    pl.kernel,
    out_type=jax.ShapeDtypeStruct((NC, NPAD), jnp.float32),
    mesh=_mesh,
    scratch_types=[
        pltpu.VMEM((NCHUNK, CH), jnp.int32),
        pltpu.VMEM((CH,), jnp.float32),
        pltpu.VMEM((RPT,), jnp.float32),
        pltpu.VMEM_SHARED((NPAD,), jnp.float32),
    ],
)
def _deg_partials(col_hbm, out_hbm, colv, onesv, zbuf, acc):
    core = lax.axis_index("c")
    sub = lax.axis_index("s")
    wid = core * NS + sub

    for k in range(RPT // 16):
        zbuf[pl.ds(16 * k, 16)] = _Z16()
    pltpu.sync_copy(zbuf, acc.at[pl.ds(sub * RPT, RPT)])
    for k in range(CH // 16):
        onesv[pl.ds(16 * k, 16)] = jnp.ones((16,), jnp.float32)
    pltpu.sync_copy(col_hbm.at[wid], colv)
    plsc.subcore_barrier()

    def body(j, carry):
        pltpu.sync_copy(onesv, acc.at[colv.at[j]], add=True)
        return carry

    lax.fori_loop(0, NCHUNK, body, 0)
    plsc.subcore_barrier()
    pltpu.sync_copy(acc.at[pl.ds(sub * RPT, RPT)],
                    out_hbm.at[core, pl.ds(sub * RPT, RPT)])


_BLK = 1000
_GRID = N // _BLK


def _prep_body(x_ref, wt_ref, degp_ref, g_ref, dis_ref):
    deg = degp_ref[:, 0:1] + degp_ref[:, 1:2] + 1.0
    dis = lax.rsqrt(deg)
    h = lax.dot_general(x_ref[...], wt_ref[...], (((1,), (0,)), ((), ())),
                        preferred_element_type=jnp.float32)
    g_ref[...] = h * dis
    dis_ref[...] = dis


def _prep(x, wt, degp_t):
    return pl.pallas_call(
        _prep_body,
        grid=(_GRID,),
        in_specs=[
            pl.BlockSpec((_BLK, D), lambda i: (i, 0)),
            pl.BlockSpec((D, D), lambda i: (0, 0)),
            pl.BlockSpec((_BLK, 2), lambda i: (i, 0)),
        ],
        out_specs=[
            pl.BlockSpec((_BLK, D), lambda i: (i, 0)),
            pl.BlockSpec((_BLK, 1), lambda i: (i, 0)),
        ],
        out_shape=[
            jax.ShapeDtypeStruct((N, D), jnp.float32),
            jax.ShapeDtypeStruct((N, 1), jnp.float32),
        ],
    )(x, wt, degp_t)


@functools.partial(
    pl.kernel,
    out_type=jax.ShapeDtypeStruct((NC, NPAD, D), jnp.float32),
    mesh=_mesh,
    scratch_types=[
        pltpu.VMEM((NCHUNK, CH), jnp.int32),
        pltpu.VMEM((NCHUNK, CH), jnp.int32),
        pltpu.VMEM((CH, D), jnp.float32),
        pltpu.VMEM_SHARED((NPAD, D), jnp.float32),
        pltpu.SemaphoreType.DMA,
    ],
)
def _edge_partials(g_hbm, row_hbm, col_hbm, out_hbm, rowv, colv, buf, acc, sem):
    core = lax.axis_index("c")
    sub = lax.axis_index("s")
    wid = core * NS + sub

    def zbody(r, carry):
        for k in range(D // 16):
            buf[r, pl.ds(16 * k, 16)] = _Z16()
        return carry

    lax.fori_loop(0, CH, zbody, 0)
    for k in range(RPT // CH):
        pltpu.sync_copy(buf, acc.at[pl.ds(sub * RPT + k * CH, CH)])

    pltpu.sync_copy(row_hbm.at[wid], rowv)
    pltpu.sync_copy(col_hbm.at[wid], colv)
    plsc.subcore_barrier()

    def body(j, carry):
        pltpu.async_copy(g_hbm.at[rowv.at[j]], buf, sem).wait()
        pltpu.sync_copy(buf, acc.at[colv.at[j]], add=True)
        return carry

    lax.fori_loop(0, NCHUNK, body, 0)
    plsc.subcore_barrier()
    pltpu.sync_copy(acc.at[pl.ds(sub * RPT, RPT)],
                    out_hbm.at[core, pl.ds(sub * RPT, RPT)])


def _final_body(s0_ref, s1_ref, g_ref, dis_ref, attn_ref, b_ref, out_ref):
    s = s0_ref[...] + s1_ref[...] + g_ref[...]
    out_ref[...] = attn_ref[...] * (dis_ref[...] * s + b_ref[...])


def _final(s0, s1, g, dis, attn, bias):
    return pl.pallas_call(
        _final_body,
        grid=(_GRID,),
        in_specs=[
            pl.BlockSpec((_BLK, D), lambda i: (i, 0)),
            pl.BlockSpec((_BLK, D), lambda i: (i, 0)),
            pl.BlockSpec((_BLK, D), lambda i: (i, 0)),
            pl.BlockSpec((_BLK, 1), lambda i: (i, 0)),
            pl.BlockSpec((_BLK, 1), lambda i: (i, 0)),
            pl.BlockSpec((1, D), lambda i: (0, 0)),
        ],
        out_specs=pl.BlockSpec((_BLK, D), lambda i: (i, 0)),
        out_shape=jax.ShapeDtypeStruct((N, D), jnp.float32),
    )(s0, s1, g, dis, attn, bias)


def kernel(x, edge_index, attention_weights, W, b):
    row = edge_index[0].reshape(NW, NCHUNK, CH)
    col = edge_index[1].reshape(NW, NCHUNK, CH)

    degp = _deg_partials(col)
    degp_t = jnp.transpose(degp)[:N, :]
    g, dis = _prep(x, jnp.transpose(W), degp_t)
    s = _edge_partials(g, row, col)[:, :N, :]
    return _final(s[0], s[1], g, dis,
                  attention_weights.reshape(N, 1), b.reshape(1, D))


# speedup vs baseline: 38.0328x; 1.4165x over previous
import functools

import jax
import jax.numpy as jnp
from jax import lax
from jax.experimental import pallas as pl
from jax.experimental.pallas import tpu as pltpu
from jax.experimental.pallas import tpu_sc as plsc

N = 10000
E = 320000
D = 128

NC = 2
NS = 16
NW = NC * NS
EPW = E // NW
CH = 80
NCHUNK = EPW // CH
NSECT = 5
SECT = NCHUNK // NSECT

NPAD = 10240
RPT = NPAD // NS

_mesh = plsc.VectorSubcoreMesh(core_axis_name="c", subcore_axis_name="s")

_Z16 = functools.partial(jnp.zeros, (16,), jnp.float32)


@functools.partial(
    pl.kernel,
    out_type=jax.ShapeDtypeStruct((NC, NPAD), jnp.float32),
    mesh=_mesh,
    scratch_types=[
        pltpu.VMEM((NCHUNK, CH), jnp.int32),
        pltpu.VMEM((CH,), jnp.float32),
        pltpu.VMEM((RPT,), jnp.float32),
        pltpu.VMEM_SHARED((NPAD,), jnp.float32),
    ],
)
def _deg_partials(col_hbm, out_hbm, colv, onesv, zbuf, acc):
    core = lax.axis_index("c")
    sub = lax.axis_index("s")
    wid = core * NS + sub

    for k in range(RPT // 16):
        zbuf[pl.ds(16 * k, 16)] = _Z16()
    pltpu.sync_copy(zbuf, acc.at[pl.ds(sub * RPT, RPT)])
    for k in range(CH // 16):
        onesv[pl.ds(16 * k, 16)] = jnp.ones((16,), jnp.float32)
    pltpu.sync_copy(col_hbm.at[wid], colv)
    plsc.subcore_barrier()

    def body(j, carry):
        pltpu.sync_copy(onesv, acc.at[colv.at[j]], add=True)
        return carry

    lax.fori_loop(0, NCHUNK, body, 0)
    plsc.subcore_barrier()
    pltpu.sync_copy(acc.at[pl.ds(sub * RPT, RPT)],
                    out_hbm.at[core, pl.ds(sub * RPT, RPT)])


_BLK = 1000
_GRID = N // _BLK


def _prep_body(x_ref, wt_ref, degp_ref, g_ref, dis_ref):
    deg = degp_ref[:, 0:1] + degp_ref[:, 1:2] + 1.0
    dis = lax.rsqrt(deg)
    h = lax.dot_general(x_ref[...], wt_ref[...], (((1,), (0,)), ((), ())),
                        preferred_element_type=jnp.float32)
    g_ref[...] = h * dis
    dis_ref[...] = dis


def _prep(x, wt, degp_t):
    return pl.pallas_call(
        _prep_body,
        grid=(_GRID,),
        in_specs=[
            pl.BlockSpec((_BLK, D), lambda i: (i, 0)),
            pl.BlockSpec((D, D), lambda i: (0, 0)),
            pl.BlockSpec((_BLK, 2), lambda i: (i, 0)),
        ],
        out_specs=[
            pl.BlockSpec((_BLK, D), lambda i: (i, 0)),
            pl.BlockSpec((_BLK, 1), lambda i: (i, 0)),
        ],
        out_shape=[
            jax.ShapeDtypeStruct((N, D), jnp.float32),
            jax.ShapeDtypeStruct((N, 1), jnp.float32),
        ],
    )(x, wt, degp_t)


@functools.partial(
    pl.kernel,
    out_type=jax.ShapeDtypeStruct((NC, NPAD, D), jnp.float32),
    mesh=_mesh,
    scratch_types=[
        pltpu.VMEM((SECT, CH), jnp.int32),
        pltpu.VMEM((SECT, CH), jnp.int32),
        pltpu.VMEM((CH, D), jnp.float32),
        pltpu.VMEM((CH, D), jnp.float32),
        pltpu.VMEM_SHARED((NPAD, D), jnp.float32),
        pltpu.SemaphoreType.DMA,
        pltpu.SemaphoreType.DMA,
    ],
)
def _edge_partials(g_hbm, row_hbm, col_hbm, out_hbm, rowv, colv, bufa, bufb,
                   acc, sema, semb):
    core = lax.axis_index("c")
    sub = lax.axis_index("s")
    wid = core * NS + sub

    def zbody(r, carry):
        for k in range(D // 16):
            bufa[r, pl.ds(16 * k, 16)] = _Z16()
        return carry

    lax.fori_loop(0, CH, zbody, 0)
    for k in range(RPT // CH):
        pltpu.sync_copy(bufa, acc.at[pl.ds(sub * RPT + k * CH, CH)])
    plsc.subcore_barrier()

    for sect in range(NSECT):
        pltpu.sync_copy(row_hbm.at[wid, sect], rowv)
        pltpu.sync_copy(col_hbm.at[wid, sect], colv)
        pltpu.async_copy(g_hbm.at[rowv.at[0]], bufa, sema)
        pltpu.async_copy(g_hbm.at[rowv.at[1]], bufb, semb)

        def body(i, carry):
            j = 2 * i
            pltpu.make_async_copy(g_hbm.at[rowv.at[j]], bufa, sema).wait()
            pltpu.sync_copy(bufa, acc.at[colv.at[j]], add=True)
            pltpu.async_copy(g_hbm.at[rowv.at[j + 2]], bufa, sema)
            pltpu.make_async_copy(g_hbm.at[rowv.at[j + 1]], bufb, semb).wait()
            pltpu.sync_copy(bufb, acc.at[colv.at[j + 1]], add=True)
            pltpu.async_copy(g_hbm.at[rowv.at[j + 3]], bufb, semb)
            return carry

        lax.fori_loop(0, (SECT - 3) // 2, body, 0)
        pltpu.make_async_copy(g_hbm.at[rowv.at[SECT - 3]], bufa, sema).wait()
        pltpu.sync_copy(bufa, acc.at[colv.at[SECT - 3]], add=True)
        pltpu.async_copy(g_hbm.at[rowv.at[SECT - 1]], bufa, sema)
        pltpu.make_async_copy(g_hbm.at[rowv.at[SECT - 2]], bufb, semb).wait()
        pltpu.sync_copy(bufb, acc.at[colv.at[SECT - 2]], add=True)
        pltpu.make_async_copy(g_hbm.at[rowv.at[SECT - 1]], bufa, sema).wait()
        pltpu.sync_copy(bufa, acc.at[colv.at[SECT - 1]], add=True)
    plsc.subcore_barrier()
    pltpu.sync_copy(acc.at[pl.ds(sub * RPT, RPT)],
                    out_hbm.at[core, pl.ds(sub * RPT, RPT)])


def _final_body(s0_ref, s1_ref, g_ref, dis_ref, attn_ref, b_ref, out_ref):
    s = s0_ref[0] + s1_ref[0] + g_ref[...]
    out_ref[...] = attn_ref[...] * (dis_ref[...] * s + b_ref[...])


def _final(s_padded, g, dis, attn, bias):
    return pl.pallas_call(
        _final_body,
        grid=(_GRID,),
        in_specs=[
            pl.BlockSpec((1, _BLK, D), lambda i: (0, i, 0)),
            pl.BlockSpec((1, _BLK, D), lambda i: (1, i, 0)),
            pl.BlockSpec((_BLK, D), lambda i: (i, 0)),
            pl.BlockSpec((_BLK, 1), lambda i: (i, 0)),
            pl.BlockSpec((_BLK, 1), lambda i: (i, 0)),
            pl.BlockSpec((1, D), lambda i: (0, 0)),
        ],
        out_specs=pl.BlockSpec((_BLK, D), lambda i: (i, 0)),
        out_shape=jax.ShapeDtypeStruct((N, D), jnp.float32),
    )(s_padded, s_padded, g, dis, attn, bias)


def kernel(x, edge_index, attention_weights, W, b):
    row = edge_index[0].reshape(NW, NSECT, SECT, CH)
    col_deg = edge_index[1].reshape(NW, NCHUNK, CH)
    col = edge_index[1].reshape(NW, NSECT, SECT, CH)

    degp = _deg_partials(col_deg)
    degp_t = jnp.transpose(degp)[:N, :]
    g, dis = _prep(x, jnp.transpose(W), degp_t)
    s = _edge_partials(g, row, col)
    return _final(s, g, dis,
                  attention_weights.reshape(N, 1), b.reshape(1, D))


# speedup vs baseline: 39.2034x; 1.0308x over previous
import functools

import jax
import jax.numpy as jnp
from jax import lax
from jax.experimental import pallas as pl
from jax.experimental.pallas import tpu as pltpu
from jax.experimental.pallas import tpu_sc as plsc

N = 10000
E = 320000
D = 128

NC = 2
NS = 16
NW = NC * NS
EPW = E // NW
CH = 80
NCHUNK = EPW // CH
SECTS = ((0, 40), (40, 40), (80, 45))
SMAX = 45

NPAD = 10240
RPT = NPAD // NS

_mesh = plsc.VectorSubcoreMesh(core_axis_name="c", subcore_axis_name="s")

_Z16 = functools.partial(jnp.zeros, (16,), jnp.float32)


@functools.partial(
    pl.kernel,
    out_type=jax.ShapeDtypeStruct((NC, NPAD), jnp.float32),
    mesh=_mesh,
    scratch_types=[
        pltpu.VMEM((NCHUNK, CH), jnp.int32),
        pltpu.VMEM((CH,), jnp.float32),
        pltpu.VMEM((RPT,), jnp.float32),
        pltpu.VMEM_SHARED((NPAD,), jnp.float32),
    ],
)
def _deg_partials(col_hbm, out_hbm, colv, onesv, zbuf, acc):
    core = lax.axis_index("c")
    sub = lax.axis_index("s")
    wid = core * NS + sub

    for k in range(RPT // 16):
        zbuf[pl.ds(16 * k, 16)] = _Z16()
    pltpu.sync_copy(zbuf, acc.at[pl.ds(sub * RPT, RPT)])
    for k in range(CH // 16):
        onesv[pl.ds(16 * k, 16)] = jnp.ones((16,), jnp.float32)
    pltpu.sync_copy(col_hbm.at[wid], colv)
    plsc.subcore_barrier()

    def body(j, carry):
        pltpu.sync_copy(onesv, acc.at[colv.at[j]], add=True)
        return carry

    lax.fori_loop(0, NCHUNK, body, 0)
    plsc.subcore_barrier()
    pltpu.sync_copy(acc.at[pl.ds(sub * RPT, RPT)],
                    out_hbm.at[core, pl.ds(sub * RPT, RPT)])


_BLK = 1000
_GRID = N // _BLK


def _matmul_body(x_ref, wt_ref, h_ref):
    h_ref[...] = lax.dot_general(x_ref[...], wt_ref[...],
                                 (((1,), (0,)), ((), ())),
                                 preferred_element_type=jnp.float32)


def _matmul(x, wt):
    return pl.pallas_call(
        _matmul_body,
        grid=(_GRID,),
        in_specs=[
            pl.BlockSpec((_BLK, D), lambda i: (i, 0)),
            pl.BlockSpec((D, D), lambda i: (0, 0)),
        ],
        out_specs=pl.BlockSpec((_BLK, D), lambda i: (i, 0)),
        out_shape=jax.ShapeDtypeStruct((N, D), jnp.float32),
    )(x, wt)


def _scale_body(h_ref, degp_ref, g_ref, dis_ref):
    deg = degp_ref[:, 0:1] + degp_ref[:, 1:2] + 1.0
    dis = lax.rsqrt(deg)
    g_ref[...] = h_ref[...] * dis
    dis_ref[...] = dis


def _scale(h, degp_t):
    return pl.pallas_call(
        _scale_body,
        grid=(_GRID,),
        in_specs=[
            pl.BlockSpec((_BLK, D), lambda i: (i, 0)),
            pl.BlockSpec((_BLK, 2), lambda i: (i, 0)),
        ],
        out_specs=[
            pl.BlockSpec((_BLK, D), lambda i: (i, 0)),
            pl.BlockSpec((_BLK, 1), lambda i: (i, 0)),
        ],
        out_shape=[
            jax.ShapeDtypeStruct((N, D), jnp.float32),
            jax.ShapeDtypeStruct((N, 1), jnp.float32),
        ],
    )(h, degp_t)


@functools.partial(
    pl.kernel,
    out_type=jax.ShapeDtypeStruct((NC, NPAD, D), jnp.float32),
    mesh=_mesh,
    scratch_types=[
        pltpu.VMEM((SMAX * CH,), jnp.int32),
        pltpu.VMEM((SMAX, CH), jnp.int32),
        pltpu.VMEM((CH, D), jnp.float32),
        pltpu.VMEM((CH, D), jnp.float32),
        pltpu.VMEM_SHARED((NPAD, D), jnp.float32),
        pltpu.SemaphoreType.DMA,
        pltpu.SemaphoreType.DMA,
    ],
)
def _edge_partials(g_hbm, row_hbm, col_hbm, out_hbm, rowv, colv, bufa, bufb,
                   acc, sema, semb):
    core = lax.axis_index("c")
    sub = lax.axis_index("s")
    wid = core * NS + sub

    def zbody(r, carry):
        for k in range(D // 16):
            bufa[r, pl.ds(16 * k, 16)] = _Z16()
        return carry

    lax.fori_loop(0, CH, zbody, 0)
    for k in range(RPT // CH):
        pltpu.sync_copy(bufa, acc.at[pl.ds(sub * RPT + k * CH, CH)])
    plsc.subcore_barrier()

    def ridx(j):
        return rowv.at[pl.ds(j * CH, CH)]

    for soff, slen in SECTS:
        pltpu.sync_copy(row_hbm.at[pl.ds((wid * NCHUNK + soff) * CH, slen * CH)],
                        rowv.at[pl.ds(0, slen * CH)])
        pltpu.sync_copy(col_hbm.at[wid, pl.ds(soff, slen)],
                        colv.at[pl.ds(0, slen)])
        pltpu.async_copy(g_hbm.at[ridx(0)], bufa, sema)
        pltpu.async_copy(g_hbm.at[ridx(1)], bufb, semb)

        def body(i, carry):
            j = 2 * i
            pltpu.make_async_copy(g_hbm.at[ridx(j)], bufa, sema).wait()
            pltpu.sync_copy(bufa, acc.at[colv.at[j]], add=True)
            pltpu.async_copy(g_hbm.at[ridx(j + 2)], bufa, sema)
            pltpu.make_async_copy(g_hbm.at[ridx(j + 1)], bufb, semb).wait()
            pltpu.sync_copy(bufb, acc.at[colv.at[j + 1]], add=True)
            pltpu.async_copy(g_hbm.at[ridx(j + 3)], bufb, semb)
            return carry

        npair = (slen - 3) // 2 if slen % 2 else (slen - 4) // 2
        lax.fori_loop(0, npair, body, 0)
        t = 2 * npair
        pltpu.make_async_copy(g_hbm.at[ridx(t)], bufa, sema).wait()
        pltpu.sync_copy(bufa, acc.at[colv.at[t]], add=True)
        if t + 2 < slen:
            pltpu.async_copy(g_hbm.at[ridx(t + 2)], bufa, sema)
        pltpu.make_async_copy(g_hbm.at[ridx(t + 1)], bufb, semb).wait()
        pltpu.sync_copy(bufb, acc.at[colv.at[t + 1]], add=True)
        if t + 3 < slen:
            pltpu.async_copy(g_hbm.at[ridx(t + 3)], bufb, semb)
        if t + 2 < slen:
            pltpu.make_async_copy(g_hbm.at[ridx(t + 2)], bufa, sema).wait()
            pltpu.sync_copy(bufa, acc.at[colv.at[t + 2]], add=True)
        if t + 3 < slen:
            pltpu.make_async_copy(g_hbm.at[ridx(t + 3)], bufb, semb).wait()
            pltpu.sync_copy(bufb, acc.at[colv.at[t + 3]], add=True)
    plsc.subcore_barrier()
    pltpu.sync_copy(acc.at[pl.ds(sub * RPT, RPT)],
                    out_hbm.at[core, pl.ds(sub * RPT, RPT)])


def _final_body(s0_ref, s1_ref, g_ref, dis_ref, attn_ref, b_ref, out_ref):
    s = s0_ref[0] + s1_ref[0] + g_ref[...]
    out_ref[...] = attn_ref[...] * (dis_ref[...] * s + b_ref[...])


def _final(s_padded, g, dis, attn, bias):
    return pl.pallas_call(
        _final_body,
        grid=(_GRID,),
        in_specs=[
            pl.BlockSpec((1, _BLK, D), lambda i: (0, i, 0)),
            pl.BlockSpec((1, _BLK, D), lambda i: (1, i, 0)),
            pl.BlockSpec((_BLK, D), lambda i: (i, 0)),
            pl.BlockSpec((_BLK, 1), lambda i: (i, 0)),
            pl.BlockSpec((_BLK, 1), lambda i: (i, 0)),
            pl.BlockSpec((1, D), lambda i: (0, 0)),
        ],
        out_specs=pl.BlockSpec((_BLK, D), lambda i: (i, 0)),
        out_shape=jax.ShapeDtypeStruct((N, D), jnp.float32),
    )(s_padded, s_padded, g, dis, attn, bias)


def kernel(x, edge_index, attention_weights, W, b):
    row = edge_index[0]
    col = edge_index[1].reshape(NW, NCHUNK, CH)

    degp = _deg_partials(col)
    h = _matmul(x, jnp.transpose(W))
    degp_t = jnp.transpose(degp)[:N, :]
    g, dis = _scale(h, degp_t)
    s = _edge_partials(g, row, col)
    return _final(s, g, dis,
                  attention_weights.reshape(N, 1), b.reshape(1, D))
